# Initial kernel scaffold; baseline (speedup 1.0000x reference)
import jax
import jax.numpy as jnp
from jax.experimental import pallas as pl


def kernel(fX, X, Z, z0, z1, z2):
    raise NotImplementedError("write your pallas kernel here")


# speedup vs baseline: 1.5733x; 1.5733x over previous
import functools

import jax
import jax.numpy as jnp
from jax import lax
from jax.experimental import pallas as pl
from jax.experimental.pallas import tpu as pltpu
from jax.experimental.pallas import tpu_sc as plsc

B = 16384

_info = plsc.get_sparse_core_info()
NC = _info.num_cores
NS = _info.num_subcores
L = _info.num_lanes
NW = NC * NS
BPW = B // NW
CH = 128
NCH = BPW // CH

_mesh = plsc.VectorSubcoreMesh(core_axis_name="c", subcore_axis_name="s")


@functools.partial(
    pl.kernel,
    out_type=jax.ShapeDtypeStruct((B,), jnp.float32),
    mesh=_mesh,
    scratch_types=[
        pltpu.VMEM((BPW,), jnp.int32),
        pltpu.VMEM((BPW,), jnp.int32),
        pltpu.VMEM((BPW,), jnp.int32),
        pltpu.VMEM((BPW,), jnp.float32),
        pltpu.VMEM((BPW,), jnp.float32),
        pltpu.VMEM((BPW,), jnp.float32),
        pltpu.VMEM((BPW,), jnp.float32),
        pltpu.SemaphoreType.DMA,
        pltpu.SemaphoreType.DMA,
    ],
)
def _gather_sum(fx_hbm, zc0_hbm, zc1_hbm, zc2_hbm, t0_hbm, t1_hbm, t2_hbm,
                out_hbm, idx0_v, idx1_v, idx2_v, g0_v, g1_v, g2_v, acc_v,
                sem_g, sem_fx):
    wid = lax.axis_index("s") * NC + lax.axis_index("c")
    base = wid * BPW

    zcols = (zc0_hbm, zc1_hbm, zc2_hbm)
    idxs = (idx0_v, idx1_v, idx2_v)
    gats = (g0_v, g1_v, g2_v)
    copies = []
    for t in range(3):
        copies.append(pltpu.async_copy(
            zcols[t].at[pl.ds(base, BPW)], idxs[t], sem_g))
    fx_copy = pltpu.async_copy(fx_hbm.at[pl.ds(base, BPW)], acc_v, sem_fx)
    for c in copies:
        c.wait()

    tables = (t0_hbm, t1_hbm, t2_hbm)
    gathers = []
    for t in range(3):
        for j in range(NCH):
            s = pl.ds(j * CH, CH)
            gathers.append(pltpu.async_copy(
                tables[t].at[idxs[t].at[s]], gats[t].at[s], sem_g))
    fx_copy.wait()
    for g in gathers:
        g.wait()

    for i in range(BPW // L):
        s = pl.ds(i * L, L)
        acc_v[s] = ((g0_v[s] + g1_v[s]) + g2_v[s]) + acc_v[s]

    pltpu.sync_copy(acc_v, out_hbm.at[pl.ds(base, BPW)])


@jax.jit
def kernel(fX, X, Z, z0, z1, z2):
    del X
    fx_flat = jnp.ravel(fX)
    zc0, zc1, zc2 = Z[:, 0], Z[:, 1], Z[:, 2]
    return _gather_sum(fx_flat, zc0, zc1, zc2, z2, z1, z0)


# speedup vs baseline: 1.5941x; 1.0132x over previous
import functools

import jax
import jax.numpy as jnp
from jax import lax
from jax.experimental import pallas as pl
from jax.experimental.pallas import tpu as pltpu
from jax.experimental.pallas import tpu_sc as plsc

B = 16384

_info = plsc.get_sparse_core_info()
NC = _info.num_cores
NS = _info.num_subcores
L = _info.num_lanes
NW = NC * NS
BPW = B // NW
CH = 512
NCH = BPW // CH

_mesh = plsc.VectorSubcoreMesh(core_axis_name="c", subcore_axis_name="s")


@functools.partial(
    pl.kernel,
    out_type=jax.ShapeDtypeStruct((B,), jnp.float32),
    mesh=_mesh,
    scratch_types=[
        pltpu.VMEM((BPW,), jnp.int32),
        pltpu.VMEM((BPW,), jnp.int32),
        pltpu.VMEM((BPW,), jnp.int32),
        pltpu.VMEM((BPW,), jnp.float32),
        pltpu.VMEM((BPW,), jnp.float32),
        pltpu.VMEM((BPW,), jnp.float32),
        pltpu.VMEM((BPW,), jnp.float32),
        pltpu.SemaphoreType.DMA,
        pltpu.SemaphoreType.DMA,
    ],
)
def _gather_sum(fx_hbm, zc0_hbm, zc1_hbm, zc2_hbm, t0_hbm, t1_hbm, t2_hbm,
                out_hbm, idx0_v, idx1_v, idx2_v, g0_v, g1_v, g2_v, acc_v,
                sem_g, sem_fx):
    wid = lax.axis_index("s") * NC + lax.axis_index("c")
    base = wid * BPW

    zcols = (zc0_hbm, zc1_hbm, zc2_hbm)
    idxs = (idx0_v, idx1_v, idx2_v)
    gats = (g0_v, g1_v, g2_v)
    copies = []
    for t in range(3):
        copies.append(pltpu.async_copy(
            zcols[t].at[pl.ds(base, BPW)], idxs[t], sem_g))
    fx_copy = pltpu.async_copy(fx_hbm.at[pl.ds(base, BPW)], acc_v, sem_fx)
    for c in copies:
        c.wait()

    tables = (t0_hbm, t1_hbm, t2_hbm)
    gathers = []
    for t in range(3):
        for j in range(NCH):
            s = pl.ds(j * CH, CH)
            gathers.append(pltpu.async_copy(
                tables[t].at[idxs[t].at[s]], gats[t].at[s], sem_g))
    fx_copy.wait()
    for g in gathers:
        g.wait()

    for i in range(BPW // L):
        s = pl.ds(i * L, L)
        acc_v[s] = ((g0_v[s] + g1_v[s]) + g2_v[s]) + acc_v[s]

    pltpu.sync_copy(acc_v, out_hbm.at[pl.ds(base, BPW)])


@jax.jit
def kernel(fX, X, Z, z0, z1, z2):
    del X
    fx_flat = jnp.ravel(fX)
    zc0, zc1, zc2 = Z[:, 0], Z[:, 1], Z[:, 2]
    return _gather_sum(fx_flat, zc0, zc1, zc2, z2, z1, z0)
